# Initial kernel scaffold; baseline (speedup 1.0000x reference)
import jax
import jax.numpy as jnp
from jax.experimental import pallas as pl


def kernel(x, edge_index, edge_weight, W, b, alpha):
    raise NotImplementedError("write your pallas kernel here")


# speedup vs baseline: 13.1468x; 13.1468x over previous
import functools

import jax
import jax.numpy as jnp
from jax import lax
from jax.experimental import pallas as pl
from jax.experimental.pallas import tpu as pltpu
from jax.experimental.pallas import tpu_sc as plsc

NC = 2
NS = 16
NW = NC * NS
L = 16
CH = 128

_MESH = dict(core_axis_name="c", subcore_axis_name="s", num_cores=NC,
             num_subcores=NS)
_SC_PARAMS = pltpu.CompilerParams(needs_layout_passes=False,
                                  use_tc_tiling_on_sc=False)


def _worker_id():
    return lax.axis_index("c") * NS + lax.axis_index("s")


def _make_deg_kernel(N, nch):
    @functools.partial(
        pl.kernel,
        out_type=jax.ShapeDtypeStruct((NW, N), jnp.float32),
        mesh=plsc.VectorSubcoreMesh(**_MESH),
        compiler_params=_SC_PARAMS,
        scratch_types=[
            pltpu.VMEM((CH,), jnp.int32),
            pltpu.VMEM((CH,), jnp.float32),
            pltpu.VMEM((N,), jnp.float32),
        ],
    )
    def deg_kernel(col_hbm, ew_hbm, out_hbm, col_v, ew_v, deg_v):
        w = _worker_id()
        zero = jnp.zeros((L,), jnp.float32)

        def zbody(i, _):
            deg_v[pl.ds(i * L, L)] = zero
            return 0

        lax.fori_loop(0, N // L, zbody, 0)

        def chunk(i, _):
            pltpu.sync_copy(col_hbm.at[w, i], col_v)
            pltpu.sync_copy(ew_hbm.at[w, i], ew_v)

            def group(g, _):
                cvec = col_v[pl.ds(g * L, L)]
                evec = ew_v[pl.ds(g * L, L)]
                plsc.addupdate_scatter(deg_v, [cvec], evec)
                return 0

            lax.fori_loop(0, CH // L, group, 0)
            return 0

        lax.fori_loop(0, nch, chunk, 0)
        pltpu.sync_copy(deg_v, out_hbm.at[w])

    return deg_kernel


def _make_spmm_kernel(N, D, nch):
    npt = N // NS

    @functools.partial(
        pl.kernel,
        out_type=jax.ShapeDtypeStruct((NC, N, D), jnp.float32),
        mesh=plsc.VectorSubcoreMesh(**_MESH),
        compiler_params=_SC_PARAMS,
        scratch_types=[
            pltpu.VMEM((CH,), jnp.int32),
            pltpu.VMEM((CH,), jnp.int32),
            pltpu.VMEM((CH,), jnp.float32),
            pltpu.VMEM((CH, D), jnp.float32),
            pltpu.VMEM_SHARED((N, D), jnp.float32),
            pltpu.SemaphoreType.DMA,
        ],
    )
    def spmm_kernel(row_hbm, col_hbm, ew_hbm, hp_hbm, out_hbm,
                    row_v, col_v, ew_v, rows_v, acc_sh, sem):
        c = lax.axis_index("c")
        s = lax.axis_index("s")
        w = c * NS + s
        zero = jnp.zeros((L,), jnp.float32)
        k_full = npt // CH
        rem = npt - k_full * CH

        def zbody(i, _):
            for j in range(D // L):
                rows_v[i, pl.ds(j * L, L)] = zero
            return 0

        lax.fori_loop(0, CH, zbody, 0)
        for k in range(k_full):
            pltpu.sync_copy(rows_v, acc_sh.at[pl.ds(s * npt + k * CH, CH)])
        if rem:
            pltpu.sync_copy(rows_v.at[pl.ds(0, rem)],
                            acc_sh.at[pl.ds(s * npt + k_full * CH, rem)])
        plsc.subcore_barrier()

        def chunk(i, _):
            pltpu.sync_copy(row_hbm.at[w, i], row_v)
            pltpu.sync_copy(col_hbm.at[w, i], col_v)
            pltpu.sync_copy(ew_hbm.at[w, i], ew_v)
            pltpu.async_copy(hp_hbm.at[row_v], rows_v, sem).wait()

            def group(g, _):
                evec = ew_v[pl.ds(g * L, L)]
                for i16 in range(L):
                    e = g * L + i16
                    scale = evec[i16]
                    for j in range(D // L):
                        sl = pl.ds(j * L, L)
                        rows_v[e, sl] = rows_v[e, sl] * scale
                return 0

            lax.fori_loop(0, CH // L, group, 0)
            pltpu.sync_copy(rows_v, acc_sh.at[col_v], add=True)
            return 0

        lax.fori_loop(0, nch, chunk, 0)
        plsc.subcore_barrier()
        pltpu.sync_copy(acc_sh.at[pl.ds(s * npt, npt)],
                        out_hbm.at[c, pl.ds(s * npt, npt)])

    return spmm_kernel


def _tc_prep(x_ref, w_ref, degp_ref, hp_ref, hself_ref, dis_ref):
    h = jnp.dot(x_ref[...], w_ref[...], preferred_element_type=jnp.float32)
    deg = 1.0 + jnp.sum(degp_ref[...], axis=0)
    dis = lax.rsqrt(deg)
    hp_ref[...] = h * dis[:, None]
    hself_ref[...] = h * (dis * dis)[:, None]
    dis_ref[...] = dis


def _tc_finish(p_ref, hself_ref, dis_ref, b_ref, alpha_ref, out_ref):
    acc = p_ref[0] + p_ref[1]
    t = acc * dis_ref[...][:, None] + hself_ref[...] + b_ref[...][None, :]
    out_ref[...] = jnp.where(t >= 0, t, alpha_ref[...][None, :] * t)


def kernel(x, edge_index, edge_weight, W, b, alpha):
    N, D = x.shape
    E = edge_weight.shape[0]

    row = edge_index[0].astype(jnp.int32)
    col = edge_index[1].astype(jnp.int32)
    ew = edge_weight.astype(jnp.float32)

    per_w = -(-E // (NW * CH)) * CH
    ep = per_w * NW
    pad = ep - E
    if pad:
        row = jnp.concatenate([row, jnp.zeros((pad,), jnp.int32)])
        col = jnp.concatenate([col, jnp.zeros((pad,), jnp.int32)])
        ew = jnp.concatenate([ew, jnp.zeros((pad,), jnp.float32)])
    nch = per_w // CH
    row3 = row.reshape(NW, nch, CH)
    col3 = col.reshape(NW, nch, CH)
    ew3 = ew.reshape(NW, nch, CH)

    degp = _make_deg_kernel(N, nch)(col3, ew3)

    hp, hself, dis = pl.pallas_call(
        _tc_prep,
        out_shape=(
            jax.ShapeDtypeStruct((N, D), jnp.float32),
            jax.ShapeDtypeStruct((N, D), jnp.float32),
            jax.ShapeDtypeStruct((N,), jnp.float32),
        ),
    )(x, W, degp)

    acc = _make_spmm_kernel(N, D, nch)(row3, col3, ew3, hp)

    out = pl.pallas_call(
        _tc_finish,
        out_shape=jax.ShapeDtypeStruct((N, D), jnp.float32),
    )(acc, hself, dis, b, alpha)
    return out
